# baseline (speedup 1.0000x reference)
import functools

import jax
import jax.numpy as jnp
from jax import lax
from jax.experimental import pallas as pl
from jax.experimental.pallas import tpu as pltpu
from jax.experimental.pallas import tpu_sc as plsc

B = 512
P = 64
K = 2048
G = 64
W = 128
L = 16
NC = 2
NW = 32
BW = B // NW


def _start_window_copies(x_hbm, win_v, b, buf, sem):
    pltpu.async_copy(x_hbm.at[b, :, pl.ds(0, W)], win_v.at[buf, 0], sem)
    pltpu.async_copy(x_hbm.at[b, :, pl.ds(K - W, W)], win_v.at[buf, 1], sem)


def _wait_window_copies(x_hbm, win_v, buf, sem):
    pltpu.make_async_copy(
        x_hbm.at[0, :, pl.ds(0, W)], win_v.at[buf, 0], sem
    ).wait()
    pltpu.make_async_copy(
        x_hbm.at[0, :, pl.ds(K - W, W)], win_v.at[buf, 1], sem
    ).wait()


UNROLL = 4


def _build_packed_indices(tau_v, pk_v):
    def body(i, _):
        t = tau_v[pl.ds(i * L, L)]
        fi = jnp.clip(t, 0, L - 1)
        bi = jnp.clip(t + L, 0, L - 1)
        sel = jnp.where(t >= 0, jnp.int32(1 << 16), jnp.int32(0))
        pk_v[pl.ds(i * L, L)] = fi | (bi << 8) | sel
        return 0

    lax.fori_loop(0, (P * G) // L, body, 0)


def _accumulate(pk_v, win_v, buf):
    def body(i, accs):
        out = list(accs)
        for u in range(UNROLL):
            p = i * UNROLL + u
            fr = win_v[buf, 0, p, pl.ds(0, L)]
            bk = win_v[buf, 1, p, pl.ds(W - L, L)]
            for j in range(G // L):
                pk = pk_v[pl.ds(p * G + j * L, L)]
                fi = pk & jnp.int32(255)
                bi = (pk >> 8) & jnp.int32(255)
                sel = pk >= jnp.int32(1 << 16)
                fv = jnp.take_along_axis(fr, fi, axis=0)
                bv = jnp.take_along_axis(bk, bi, axis=0)
                out[j] = out[j] + jnp.where(sel, fv, bv)
        return tuple(out)

    zero = jnp.zeros((L,), jnp.float32)
    return lax.fori_loop(0, P // UNROLL, body, (zero,) * (G // L))


def _normalize_store(accs, outbuf_v, b_local):
    mx = accs[0]
    for a in accs[1:]:
        mx = jnp.maximum(mx, a)
    lane = lax.iota(jnp.int32, L)
    for s in (8, 4, 2, 1):
        mx = jnp.maximum(mx, jnp.take_along_axis(mx, lane ^ s, axis=0))
    m = mx + jnp.float32(1e-12)
    for j in range(G // L):
        outbuf_v[b_local, pl.ds(j * L, L)] = (accs[j] + jnp.float32(1e-12)) / m


def _srp_sc_kernel(
    x_hbm, tau0_hbm, out_hbm, tau_v, pk_v, win_v, outbuf_v, sem0, sem1
):
    wid = lax.axis_index("s") * NC + lax.axis_index("c")
    base = wid * BW

    pltpu.sync_copy(tau0_hbm, tau_v)
    _build_packed_indices(tau_v, pk_v)

    _start_window_copies(x_hbm, win_v, base, 0, sem0)

    def pair(i, carry):
        b_even = base + 2 * i

        _start_window_copies(x_hbm, win_v, b_even + 1, 1, sem1)
        _wait_window_copies(x_hbm, win_v, 0, sem0)
        accs = _accumulate(pk_v, win_v, 0)
        _normalize_store(accs, outbuf_v, 2 * i)

        @pl.when(i < (BW // 2) - 1)
        def _prefetch():
            _start_window_copies(x_hbm, win_v, b_even + 2, 0, sem0)

        _wait_window_copies(x_hbm, win_v, 1, sem1)
        accs = _accumulate(pk_v, win_v, 1)
        _normalize_store(accs, outbuf_v, 2 * i + 1)
        return carry

    lax.fori_loop(0, BW // 2, pair, 0)

    pltpu.sync_copy(outbuf_v, out_hbm.at[pl.ds(base, BW), :])


@jax.jit
def kernel(x, tau0):
    xr = x.reshape(B, P, K)
    t0 = tau0.reshape(P * G)

    mesh = plsc.VectorSubcoreMesh(core_axis_name="c", subcore_axis_name="s")
    run = functools.partial(
        pl.kernel,
        mesh=mesh,
        out_type=jax.ShapeDtypeStruct((B, G), jnp.float32),
        scratch_types=[
            pltpu.VMEM((P * G,), jnp.int32),
            pltpu.VMEM((P * G,), jnp.int32),
            pltpu.VMEM((2, 2, P, W), jnp.float32),
            pltpu.VMEM((BW, G), jnp.float32),
            pltpu.SemaphoreType.DMA,
            pltpu.SemaphoreType.DMA,
        ],
    )(_srp_sc_kernel)
    return run(xr, t0)


# speedup vs baseline: 1.4582x; 1.4582x over previous
import jax
import jax.numpy as jnp
from jax.experimental import pallas as pl
from jax.experimental.pallas import tpu as pltpu

B = 512
P = 64
K = 2048
G = 64
W = 128


def _srp_tc_kernel(tau0_ref, front_ref, back_ref, out_ref, oh_ref):
    Bb = front_ref.shape[0]

    @pl.when(pl.program_id(0) == 0)
    def _build_onehot():
        idx = tau0_ref[...]
        idx = jnp.where(idx < 0, idx + 2 * W, idx)
        iota = jax.lax.broadcasted_iota(jnp.int32, (2 * W, G), 0)
        for p in range(P):
            oh_ref[p, :, :] = (iota == idx[p : p + 1, :]).astype(jnp.float32)

    acc = jnp.zeros((Bb, G), dtype=jnp.float32)
    for p in range(P):
        xw = jnp.concatenate([front_ref[:, p, :], back_ref[:, p, :]], axis=-1)
        acc += jnp.dot(xw, oh_ref[p, :, :], preferred_element_type=jnp.float32)
    maps = acc + 1e-12
    out_ref[...] = maps / jnp.max(maps, axis=-1, keepdims=True)


@jax.jit
def kernel(x, tau0):
    xr = x.reshape(B, P, K)
    t0 = tau0.reshape(P, G)

    Bb = 64
    grid = (B // Bb,)
    return pl.pallas_call(
        _srp_tc_kernel,
        grid=grid,
        in_specs=[
            pl.BlockSpec((P, G), lambda i: (0, 0)),
            pl.BlockSpec((Bb, P, W), lambda i: (i, 0, 0)),
            pl.BlockSpec((Bb, P, W), lambda i: (i, 0, K // W - 1)),
        ],
        out_specs=pl.BlockSpec((Bb, G), lambda i: (i, 0)),
        out_shape=jax.ShapeDtypeStruct((B, G), jnp.float32),
        scratch_shapes=[pltpu.VMEM((P, 2 * W, G), jnp.float32)],
        compiler_params=pltpu.CompilerParams(
            dimension_semantics=("arbitrary",),
        ),
    )(t0, xr, xr)
